# Initial kernel scaffold; baseline (speedup 1.0000x reference)
import jax
import jax.numpy as jnp
from jax.experimental import pallas as pl


def kernel(x, edge_index, edge_attr, batch, We, W1, b1, W2, b2):
    raise NotImplementedError("write your pallas kernel here")


# speedup vs baseline: 2.2114x; 2.2114x over previous
import functools
import jax
import jax.numpy as jnp
from jax import lax
from jax.experimental import pallas as pl
from jax.experimental.pallas import tpu as pltpu
from jax.experimental.pallas import tpu_sc as plsc

DIM = 256
HALF = 128
D_EDGE = 16
NUM_LAYERS = 3
N_NODES = 10000
N_EDGES = 160000
N_GRAPHS = 128

NC = 2
NS = 16
CHUNK = 128
NCHUNKS = N_EDGES // CHUNK
ROWS_PER_TILE = 632
N_PAD = NS * ROWS_PER_TILE

EB = 4000
NB = 1000


def _edge_emb_body(ea_ref, we_ref, e0_ref, e1_ref, e2_ref):
    ea = ea_ref[...]
    for i, out in enumerate((e0_ref, e1_ref, e2_ref)):
        e = jnp.dot(ea, we_ref[i], preferred_element_type=jnp.float32)
        out[0] = e[:, :HALF]
        out[1] = e[:, HALF:]


def _edge_embeddings(edge_attr, We):
    eshape = jax.ShapeDtypeStruct((NC, N_EDGES, HALF), jnp.float32)
    return pl.pallas_call(
        _edge_emb_body,
        grid=(N_EDGES // EB,),
        in_specs=[
            pl.BlockSpec((EB, D_EDGE), lambda b: (b, 0)),
            pl.BlockSpec((NUM_LAYERS, D_EDGE, DIM), lambda b: (0, 0, 0)),
        ],
        out_specs=[
            pl.BlockSpec((NC, EB, HALF), lambda b: (0, b, 0)),
            pl.BlockSpec((NC, EB, HALF), lambda b: (0, b, 0)),
            pl.BlockSpec((NC, EB, HALF), lambda b: (0, b, 0)),
        ],
        out_shape=[eshape, eshape, eshape],
    )(edge_attr, We)


def _sc_aggr_body(h2_hbm, src_hbm, dst_hbm, e2_hbm, out_hbm,
                  srcv, dstv, hrows, mrows, acc, sem):
    c = lax.axis_index("c")
    s = lax.axis_index("s")

    def _zero_row(i, _):
        for v in range(HALF // 16):
            hrows[i, pl.ds(v * 16, 16)] = jnp.zeros((16,), jnp.float32)
        return 0
    lax.fori_loop(0, CHUNK, _zero_row, 0)
    for j in range(4):
        pltpu.sync_copy(hrows,
                        acc.at[pl.ds(s * ROWS_PER_TILE + j * CHUNK, CHUNK)])
    pltpu.sync_copy(hrows.at[pl.ds(0, 120)],
                    acc.at[pl.ds(s * ROWS_PER_TILE + 4 * CHUNK, 120)])
    plsc.subcore_barrier()

    nj = jnp.where(s < NCHUNKS % NS, NCHUNKS // NS + 1, NCHUNKS // NS)

    def _chunk(j, _):
        cid = s + j * NS
        base = cid * CHUNK
        pltpu.sync_copy(src_hbm.at[pl.ds(base, CHUNK)], srcv)
        pltpu.sync_copy(dst_hbm.at[pl.ds(base, CHUNK)], dstv)
        for v in range(CHUNK // 16):
            sl = pl.ds(v * 16, 16)
            srcv[sl] = srcv[sl] * 2 + c
        pltpu.async_copy(h2_hbm.at[srcv], hrows, sem).wait()
        pltpu.sync_copy(e2_hbm.at[pl.ds(c * N_EDGES + base, CHUNK)], mrows)

        def _msg(i, _):
            for v in range(HALF // 16):
                sl = pl.ds(v * 16, 16)
                mrows[i, sl] = jnp.maximum(hrows[i, sl] + mrows[i, sl], 0.0)
            return 0
        lax.fori_loop(0, CHUNK, _msg, 0)
        pltpu.sync_copy(mrows, acc.at[dstv], add=True)
        return 0

    lax.fori_loop(0, nj, _chunk, 0)
    plsc.subcore_barrier()

    pltpu.sync_copy(
        acc.at[pl.ds(s * ROWS_PER_TILE, ROWS_PER_TILE)],
        out_hbm.at[pl.ds(c * N_PAD + s * ROWS_PER_TILE, ROWS_PER_TILE)])


_sc_aggr = pl.kernel(
    _sc_aggr_body,
    out_type=jax.ShapeDtypeStruct((NC * N_PAD, HALF), jnp.float32),
    mesh=plsc.VectorSubcoreMesh(core_axis_name="c", subcore_axis_name="s"),
    scratch_types=[
        pltpu.VMEM((CHUNK,), jnp.int32),
        pltpu.VMEM((CHUNK,), jnp.int32),
        pltpu.VMEM((CHUNK, HALF), jnp.float32),
        pltpu.VMEM((CHUNK, HALF), jnp.float32),
        pltpu.VMEM_SHARED((N_PAD, HALF), jnp.float32),
        pltpu.SemaphoreType.DMA,
    ],
)


def _mlp_body(h_ref, a_ref, batch_ref, w1_ref, b1_ref, w2_ref, b2_ref,
              out_ref, pool_ref):
    u = h_ref[...] + jnp.concatenate([a_ref[0], a_ref[1]], axis=1)
    z = jnp.maximum(
        jnp.dot(u, w1_ref[...], preferred_element_type=jnp.float32)
        + b1_ref[...], 0.0)
    o = jnp.dot(z, w2_ref[...], preferred_element_type=jnp.float32) \
        + b2_ref[...]
    hn = jnp.maximum(o, 0.0)
    out_ref[...] = hn

    b = batch_ref[0, 0, :]
    onehot = (b[None, :]
              == lax.broadcasted_iota(jnp.int32, (N_GRAPHS, NB), 0)
              ).astype(jnp.float32)

    @pl.when(pl.program_id(0) == 0)
    def _():
        pool_ref[...] = jnp.zeros_like(pool_ref)
    pool_ref[...] += jnp.dot(onehot, hn, preferred_element_type=jnp.float32)


def _mlp_pool(h, aggr, batm, W1i, b1i, W2i, b2i):
    return pl.pallas_call(
        _mlp_body,
        grid=(N_NODES // NB,),
        in_specs=[
            pl.BlockSpec((NB, DIM), lambda b: (b, 0)),
            pl.BlockSpec((NC, NB, HALF), lambda b: (0, b, 0)),
            pl.BlockSpec((1, 1, NB), lambda b: (b, 0, 0)),
            pl.BlockSpec((DIM, 2 * DIM), lambda b: (0, 0)),
            pl.BlockSpec((1, 2 * DIM), lambda b: (0, 0)),
            pl.BlockSpec((2 * DIM, DIM), lambda b: (0, 0)),
            pl.BlockSpec((1, DIM), lambda b: (0, 0)),
        ],
        out_specs=[
            pl.BlockSpec((NB, DIM), lambda b: (b, 0)),
            pl.BlockSpec((N_GRAPHS, DIM), lambda b: (0, 0)),
        ],
        out_shape=[
            jax.ShapeDtypeStruct((N_NODES, DIM), jnp.float32),
            jax.ShapeDtypeStruct((N_GRAPHS, DIM), jnp.float32),
        ],
    )(h, aggr, batm, W1i, b1i, W2i, b2i)


def kernel(x, edge_index, edge_attr, batch, We, W1, b1, W2, b2):
    src = edge_index[0]
    dst = edge_index[1]
    es = _edge_embeddings(edge_attr, We)
    batm = batch.reshape(N_NODES // NB, 1, NB)

    h = x
    hs = []
    pools = []
    for i in range(NUM_LAYERS):
        h2 = h.reshape(2 * N_NODES, HALF)
        e2 = es[i].reshape(NC * N_EDGES, HALF)
        aggr2 = _sc_aggr(h2, src, dst, e2)
        aggr = aggr2.reshape(NC, N_PAD, HALF)[:, :N_NODES, :]
        h, pool = _mlp_pool(h, aggr, batm, W1[i], b1[i].reshape(1, -1),
                            W2[i], b2[i].reshape(1, -1))
        hs.append(h)
        pools.append(pool)

    return (jnp.concatenate(pools, axis=1), jnp.concatenate(hs, axis=1))


# speedup vs baseline: 3.6866x; 1.6670x over previous
import functools
import jax
import jax.numpy as jnp
from jax import lax
from jax.experimental import pallas as pl
from jax.experimental.pallas import tpu as pltpu
from jax.experimental.pallas import tpu_sc as plsc

DIM = 256
HALF = 128
D_EDGE = 16
NUM_LAYERS = 3
N_NODES = 10000
N_EDGES = 160000
N_GRAPHS = 128

NC = 2
NS = 16
CHUNK = 64
NCHUNKS = N_EDGES // CHUNK
NJT = 159
ROWS_PER_TILE = 632
N_PAD = NS * ROWS_PER_TILE

EB = 4000
NB = 1000


def _edge_emb_body(ea_ref, we_ref, e_ref):
    e = jnp.dot(ea_ref[...], we_ref[0],
                preferred_element_type=jnp.float32)
    e_ref[0] = e[:, :HALF]
    e_ref[1] = e[:, HALF:]


def _edge_embeddings(edge_attr, Wei):
    return pl.pallas_call(
        _edge_emb_body,
        grid=(N_EDGES // EB,),
        in_specs=[
            pl.BlockSpec((EB, D_EDGE), lambda b: (b, 0)),
            pl.BlockSpec((1, D_EDGE, DIM), lambda b: (0, 0, 0)),
        ],
        out_specs=pl.BlockSpec((NC, EB, HALF), lambda b: (0, b, 0)),
        out_shape=jax.ShapeDtypeStruct((NC, N_EDGES, HALF), jnp.float32),
    )(edge_attr, Wei)


def _sc_aggr_body(h2_hbm, src_hbm, dst_hbm, e2_hbm, out_hbm,
                  srcv, dstv, hrows, mrows, acc,
                  isem, gsem, esem, ssem):
    c = lax.axis_index("c")
    s = lax.axis_index("s")

    @plsc.parallel_loop(0, CHUNK, step=1, unroll=4)
    def _zero_row(i):
        for v in range(HALF // 16):
            hrows[0, i, pl.ds(v * 16, 16)] = jnp.zeros((16,), jnp.float32)
    for j in range(9):
        pltpu.sync_copy(hrows.at[0],
                        acc.at[pl.ds(s * ROWS_PER_TILE + j * CHUNK, CHUNK)])
    pltpu.sync_copy(hrows.at[0, pl.ds(0, 56)],
                    acc.at[pl.ds(s * ROWS_PER_TILE + 9 * CHUNK, 56)])
    plsc.subcore_barrier()

    def _base(k):
        cid = s + k * NS
        valid = cid < NCHUNKS
        return valid, pl.multiple_of(jnp.where(valid, cid, 0) * CHUNK, CHUNK)

    def _fire_idx(k, pb):
        _, base = _base(k)
        pltpu.async_copy(src_hbm.at[pl.ds(base, CHUNK)], srcv.at[pb],
                         isem[pb])
        pltpu.async_copy(dst_hbm.at[pl.ds(base, CHUNK)], dstv.at[pb],
                         isem[pb])

    def _wait_idx(pb):
        pltpu.make_async_copy(src_hbm.at[pl.ds(0, CHUNK)], srcv.at[pb],
                              isem[pb]).wait()
        pltpu.make_async_copy(dst_hbm.at[pl.ds(0, CHUNK)], dstv.at[pb],
                              isem[pb]).wait()

    def _fire_rows(k, pb):
        valid, base = _base(k)
        for v in range(CHUNK // 16):
            sl = pl.ds(v * 16, 16)
            srcv[pb, sl] = srcv[pb, sl] * 2 + c
            dstv[pb, sl] = jnp.where(valid, dstv[pb, sl], N_NODES)
        pltpu.async_copy(h2_hbm.at[srcv.at[pb]], hrows.at[pb], gsem[pb])
        erow = pl.multiple_of(c * N_EDGES + base, CHUNK)
        pltpu.async_copy(e2_hbm.at[pl.ds(erow, CHUNK)],
                         mrows.at[pb], esem[pb])

    _fire_idx(0, 0)
    _wait_idx(0)
    _fire_rows(0, 0)
    _fire_idx(1, 1)

    NJJ = NJT // 3

    def _step(jj, _):
        for p in range(3):
            k = 3 * jj + p
            b = p
            pb = (p + 1) % 3
            ppb = (p + 2) % 3

            def _next_rows():
                _wait_idx(pb)
                _fire_rows(k + 1, pb)
            if p == 2:
                pl.when(jj < NJJ - 1)(_next_rows)
            else:
                _next_rows()

            pltpu.make_async_copy(h2_hbm.at[srcv.at[b]], hrows.at[b],
                                  gsem[b]).wait()
            pltpu.make_async_copy(e2_hbm.at[pl.ds(0, CHUNK)], mrows.at[b],
                                  esem[b]).wait()

            @plsc.parallel_loop(0, CHUNK, step=1, unroll=4)
            def _msg(i):
                for v in range(HALF // 16):
                    sl = pl.ds(v * 16, 16)
                    mrows[b, i, sl] = jnp.maximum(
                        hrows[b, i, sl] + mrows[b, i, sl], 0.0)

            pltpu.async_copy(mrows.at[b], acc.at[dstv.at[b]], ssem[b],
                             add=True)

            def _drain_scatter():
                pltpu.make_async_copy(
                    mrows.at[ppb], acc.at[dstv.at[ppb]], ssem[ppb]).wait()
            if p == 0:
                pl.when(jj >= 1)(_drain_scatter)
            else:
                _drain_scatter()

            def _next_idx():
                _fire_idx(k + 2, ppb)
            if p == 0:
                _next_idx()
            else:
                pl.when(jj < NJJ - 1)(_next_idx)
        return 0

    lax.fori_loop(0, NJJ, _step, 0)
    pltpu.make_async_copy(mrows.at[2], acc.at[dstv.at[2]], ssem[2]).wait()
    plsc.subcore_barrier()

    pltpu.sync_copy(
        acc.at[pl.ds(s * ROWS_PER_TILE, ROWS_PER_TILE)],
        out_hbm.at[pl.ds(c * N_PAD + s * ROWS_PER_TILE, ROWS_PER_TILE)])


_sc_aggr = pl.kernel(
    _sc_aggr_body,
    out_type=jax.ShapeDtypeStruct((NC * N_PAD, HALF), jnp.float32),
    mesh=plsc.VectorSubcoreMesh(core_axis_name="c", subcore_axis_name="s"),
    scratch_types=[
        pltpu.VMEM((3, CHUNK), jnp.int32),
        pltpu.VMEM((3, CHUNK), jnp.int32),
        pltpu.VMEM((3, CHUNK, HALF), jnp.float32),
        pltpu.VMEM((3, CHUNK, HALF), jnp.float32),
        pltpu.VMEM_SHARED((N_PAD, HALF), jnp.float32),
        [pltpu.SemaphoreType.DMA] * 3,
        [pltpu.SemaphoreType.DMA] * 3,
        [pltpu.SemaphoreType.DMA] * 3,
        [pltpu.SemaphoreType.DMA] * 3,
    ],
)


def _mlp_body(h_ref, a_ref, batch_ref, w1_ref, b1_ref, w2_ref, b2_ref,
              out_ref, pool_ref):
    u = h_ref[...] + jnp.concatenate([a_ref[0], a_ref[1]], axis=1)
    z = jnp.maximum(
        jnp.dot(u, w1_ref[...], preferred_element_type=jnp.float32)
        + b1_ref[...], 0.0)
    o = jnp.dot(z, w2_ref[...], preferred_element_type=jnp.float32) \
        + b2_ref[...]
    hn = jnp.maximum(o, 0.0)
    out_ref[...] = hn

    b = batch_ref[0, 0, :]
    onehot = (b[None, :]
              == lax.broadcasted_iota(jnp.int32, (N_GRAPHS, NB), 0)
              ).astype(jnp.float32)

    @pl.when(pl.program_id(0) == 0)
    def _():
        pool_ref[...] = jnp.zeros_like(pool_ref)
    pool_ref[...] += jnp.dot(onehot, hn, preferred_element_type=jnp.float32)


def _mlp_pool(h, aggr, batm, W1i, b1i, W2i, b2i):
    return pl.pallas_call(
        _mlp_body,
        grid=(N_NODES // NB,),
        in_specs=[
            pl.BlockSpec((NB, DIM), lambda b: (b, 0)),
            pl.BlockSpec((NC, NB, HALF), lambda b: (0, b, 0)),
            pl.BlockSpec((1, 1, NB), lambda b: (b, 0, 0)),
            pl.BlockSpec((DIM, 2 * DIM), lambda b: (0, 0)),
            pl.BlockSpec((1, 2 * DIM), lambda b: (0, 0)),
            pl.BlockSpec((2 * DIM, DIM), lambda b: (0, 0)),
            pl.BlockSpec((1, DIM), lambda b: (0, 0)),
        ],
        out_specs=[
            pl.BlockSpec((NB, DIM), lambda b: (b, 0)),
            pl.BlockSpec((N_GRAPHS, DIM), lambda b: (0, 0)),
        ],
        out_shape=[
            jax.ShapeDtypeStruct((N_NODES, DIM), jnp.float32),
            jax.ShapeDtypeStruct((N_GRAPHS, DIM), jnp.float32),
        ],
    )(h, aggr, batm, W1i, b1i, W2i, b2i)


def kernel(x, edge_index, edge_attr, batch, We, W1, b1, W2, b2):
    batm = batch.reshape(N_NODES // NB, 1, NB)

    h = x
    hs = []
    pools = []
    for i in range(NUM_LAYERS):
        e = _edge_embeddings(edge_attr, We[i:i + 1])
        h2 = h.reshape(2 * N_NODES, HALF)
        e2 = e.reshape(NC * N_EDGES, HALF)
        aggr2 = _sc_aggr(h2, edge_index[0], edge_index[1], e2)
        aggr = aggr2.reshape(NC, N_PAD, HALF)[:, :N_NODES, :]
        h, pool = _mlp_pool(h, aggr, batm, W1[i], b1[i].reshape(1, -1),
                            W2[i], b2[i].reshape(1, -1))
        hs.append(h)
        pools.append(pool)

    return (jnp.concatenate(pools, axis=1), jnp.concatenate(hs, axis=1))
